# baseline (speedup 1.0000x reference)
import functools

import jax
import jax.numpy as jnp
from jax.experimental import pallas as pl


def _router_block(x_ref, w1_ref, b1_ref, w2_ref, b2_ref,
                  logits_ref, weights_ref, idx_ref):
    x = x_ref[...].astype(jnp.bfloat16)
    h = jnp.dot(x, w1_ref[...], preferred_element_type=jnp.float32)
    h = jnp.maximum(h + b1_ref[...], 0.0)
    logits = jnp.dot(h.astype(jnp.bfloat16), w2_ref[...],
                     preferred_element_type=jnp.float32)
    logits = logits + b2_ref[...]
    logits_ref[...] = logits

    m = jnp.max(logits, axis=1, keepdims=True)
    e = jnp.exp(logits - m)
    probs = e / jnp.sum(e, axis=1, keepdims=True)

    t = probs.shape[0]
    iota = jax.lax.broadcasted_iota(jnp.int32, (t, 16), 1)
    m1 = jnp.max(probs, axis=1, keepdims=True)
    i1 = jnp.min(jnp.where(probs == m1, iota, 16), axis=1, keepdims=True)
    probs2 = jnp.where(iota == i1, -1.0, probs)
    m2 = jnp.max(probs2, axis=1, keepdims=True)
    i2 = jnp.min(jnp.where(probs2 == m2, iota, 16), axis=1, keepdims=True)

    weights_ref[...] = jnp.concatenate([m1, m2], axis=1)
    idx_ref[...] = jnp.concatenate([i1, i2], axis=1)


@functools.partial(jax.jit, static_argnames=("block_t",))
def _router(hidden_states, W1, b1, W2, b2, block_t=512):
    b, s, hdim = hidden_states.shape
    n_tok = b * s
    x = hidden_states.reshape(n_tok, hdim)
    half = W1.shape[1]
    ne = W2.shape[1]
    grid = (n_tok // block_t,)

    logits, weights, idx = pl.pallas_call(
        _router_block,
        grid=grid,
        in_specs=[
            pl.BlockSpec((block_t, hdim), lambda i: (i, 0)),
            pl.BlockSpec((hdim, half), lambda i: (0, 0)),
            pl.BlockSpec((1, half), lambda i: (0, 0)),
            pl.BlockSpec((half, ne), lambda i: (0, 0)),
            pl.BlockSpec((1, ne), lambda i: (0, 0)),
        ],
        out_specs=[
            pl.BlockSpec((block_t, ne), lambda i: (i, 0)),
            pl.BlockSpec((block_t, 2), lambda i: (i, 0)),
            pl.BlockSpec((block_t, 2), lambda i: (i, 0)),
        ],
        out_shape=[
            jax.ShapeDtypeStruct((n_tok, ne), jnp.float32),
            jax.ShapeDtypeStruct((n_tok, 2), jnp.float32),
            jax.ShapeDtypeStruct((n_tok, 2), jnp.int32),
        ],
    )(x, W1.astype(jnp.bfloat16), b1.reshape(1, half),
      W2.astype(jnp.bfloat16), b2.reshape(1, ne))

    return (logits.reshape(b, s, ne),
            weights.reshape(b, s, 2),
            idx.reshape(b, s, 2))


def kernel(hidden_states, W1, b1, W2, b2):
    return _router(hidden_states, W1, b1, W2, b2)


# speedup vs baseline: 1.4697x; 1.4697x over previous
import functools

import jax
import jax.numpy as jnp
from jax.experimental import pallas as pl


def _router_block(x_ref, w1_ref, b1_ref, w2t_ref, b2t_ref,
                  logits_ref, weights_ref, idx_ref):
    x = x_ref[...].astype(jnp.bfloat16)
    h = jnp.dot(x, w1_ref[...], preferred_element_type=jnp.float32)
    h = jnp.maximum(h + b1_ref[...], 0.0)
    lt = jax.lax.dot_general(
        w2t_ref[...], h.astype(jnp.bfloat16),
        dimension_numbers=(((1,), (1,)), ((), ())),
        preferred_element_type=jnp.float32)
    lt = lt + b2t_ref[...]
    logits_ref[...] = lt

    ne, t = lt.shape
    m = jnp.max(lt, axis=0, keepdims=True)
    e = jnp.exp(lt - m)
    s = jnp.sum(e, axis=0, keepdims=True)

    iota = jax.lax.broadcasted_iota(jnp.int32, (ne, t), 0)
    i1 = jnp.min(jnp.where(lt == m, iota, ne), axis=0, keepdims=True)
    lt2 = jnp.where(iota == i1, -jnp.inf, lt)
    m2 = jnp.max(lt2, axis=0, keepdims=True)
    i2 = jnp.min(jnp.where(lt2 == m2, iota, ne), axis=0, keepdims=True)

    w1 = 1.0 / s
    w2 = jnp.exp(m2 - m) / s
    weights_ref[...] = jnp.concatenate([w1, w2], axis=0)
    idx_ref[...] = jnp.concatenate([i1, i2], axis=0)


@functools.partial(jax.jit, static_argnames=("block_t",))
def _router(hidden_states, W1, b1, W2, b2, block_t=1024):
    b, s, hdim = hidden_states.shape
    n_tok = b * s
    x = hidden_states.reshape(n_tok, hdim)
    half = W1.shape[1]
    ne = W2.shape[1]
    grid = (n_tok // block_t,)

    logits_t, weights_t, idx_t = pl.pallas_call(
        _router_block,
        grid=grid,
        in_specs=[
            pl.BlockSpec((block_t, hdim), lambda i: (i, 0)),
            pl.BlockSpec((hdim, half), lambda i: (0, 0)),
            pl.BlockSpec((1, half), lambda i: (0, 0)),
            pl.BlockSpec((ne, half), lambda i: (0, 0)),
            pl.BlockSpec((ne, 1), lambda i: (0, 0)),
        ],
        out_specs=[
            pl.BlockSpec((ne, block_t), lambda i: (0, i)),
            pl.BlockSpec((2, block_t), lambda i: (0, i)),
            pl.BlockSpec((2, block_t), lambda i: (0, i)),
        ],
        out_shape=[
            jax.ShapeDtypeStruct((ne, n_tok), jnp.float32),
            jax.ShapeDtypeStruct((2, n_tok), jnp.float32),
            jax.ShapeDtypeStruct((2, n_tok), jnp.int32),
        ],
    )(x, W1.astype(jnp.bfloat16), b1.reshape(1, half),
      W2.T.astype(jnp.bfloat16), b2.reshape(ne, 1))

    return (logits_t.T.reshape(b, s, ne),
            weights_t.T.reshape(b, s, 2),
            idx_t.T.reshape(b, s, 2))


def kernel(hidden_states, W1, b1, W2, b2):
    return _router(hidden_states, W1, b1, W2, b2)


# speedup vs baseline: 1.5726x; 1.0700x over previous
import functools

import jax
import jax.numpy as jnp
from jax.experimental import pallas as pl


def _router_block(x_ref, w1_ref, b1_ref, w2t_ref, b2t_ref,
                  logits_ref, weights_ref, idx_ref):
    h = jnp.dot(x_ref[...], w1_ref[...], preferred_element_type=jnp.float32)
    h = jnp.maximum(h + b1_ref[...], 0.0)
    lt = jax.lax.dot_general(
        w2t_ref[...], h,
        dimension_numbers=(((1,), (1,)), ((), ())),
        preferred_element_type=jnp.float32)
    lt = lt + b2t_ref[...]
    logits_ref[...] = lt

    ne, t = lt.shape
    m = jnp.max(lt, axis=0, keepdims=True)
    e = jnp.exp(lt - m)
    s = jnp.sum(e, axis=0, keepdims=True)

    iota = jax.lax.broadcasted_iota(jnp.int32, (ne, t), 0)
    i1 = jnp.min(jnp.where(lt == m, iota, ne), axis=0, keepdims=True)
    lt2 = jnp.where(iota == i1, -jnp.inf, lt)
    m2 = jnp.max(lt2, axis=0, keepdims=True)
    i2 = jnp.min(jnp.where(lt2 == m2, iota, ne), axis=0, keepdims=True)

    w1 = 1.0 / s
    w2 = jnp.exp(m2 - m) / s
    weights_ref[...] = jnp.concatenate([w1, w2], axis=0)
    idx_ref[...] = jnp.concatenate([i1, i2], axis=0)


@functools.partial(jax.jit, static_argnames=("block_t",))
def _router(hidden_states, W1, b1, W2, b2, block_t=1024):
    b, s, hdim = hidden_states.shape
    n_tok = b * s
    x = hidden_states.reshape(n_tok, hdim)
    half = W1.shape[1]
    ne = W2.shape[1]
    grid = (n_tok // block_t,)

    logits_t, weights_t, idx_t = pl.pallas_call(
        _router_block,
        grid=grid,
        in_specs=[
            pl.BlockSpec((block_t, hdim), lambda i: (i, 0)),
            pl.BlockSpec((hdim, half), lambda i: (0, 0)),
            pl.BlockSpec((1, half), lambda i: (0, 0)),
            pl.BlockSpec((ne, half), lambda i: (0, 0)),
            pl.BlockSpec((ne, 1), lambda i: (0, 0)),
        ],
        out_specs=[
            pl.BlockSpec((ne, block_t), lambda i: (0, i)),
            pl.BlockSpec((2, block_t), lambda i: (0, i)),
            pl.BlockSpec((2, block_t), lambda i: (0, i)),
        ],
        out_shape=[
            jax.ShapeDtypeStruct((ne, n_tok), jnp.float32),
            jax.ShapeDtypeStruct((2, n_tok), jnp.float32),
            jax.ShapeDtypeStruct((2, n_tok), jnp.int32),
        ],
    )(x, W1, b1.reshape(1, half), W2.T, b2.reshape(ne, 1))

    return (logits_t.T.reshape(b, s, ne),
            weights_t.T.reshape(b, s, 2),
            idx_t.T.reshape(b, s, 2))


def kernel(hidden_states, W1, b1, W2, b2):
    return _router(hidden_states, W1, b1, W2, b2)
